# Initial kernel scaffold; baseline (speedup 1.0000x reference)
import jax
import jax.numpy as jnp
from jax.experimental import pallas as pl


def kernel(x, edge_index, neg_src, neg_dst, W1, b1, W2, b2, W3, b3, Wlp, blp):
    raise NotImplementedError("write your pallas kernel here")


# speedup vs baseline: 5.9852x; 5.9852x over previous
import functools

import jax
import jax.numpy as jnp
from jax import lax
from jax.experimental import pallas as pl
from jax.experimental.pallas import tpu as pltpu
from jax.experimental.pallas import tpu_sc as plsc

N = 10000
E = 320000
D = 128
NPAD = 10240
NC = 2
NS = 16
EPC = E // NC
EPT = EPC // NS
CH = 128
NFULL = EPT // CH
TAIL = EPT - NFULL * CH

_mesh = plsc.VectorSubcoreMesh(core_axis_name="c", subcore_axis_name="s")


def _fill(ref, n, value):
    def body(i, _):
        ref[pl.ds(pl.multiple_of(i * 16, 16), 16)] = jnp.full((16,), value, jnp.float32)
        return 0
    lax.fori_loop(0, n // 16, body, 0)


@functools.partial(
    pl.kernel,
    out_type=jax.ShapeDtypeStruct((2 * NPAD,), jnp.float32),
    mesh=_mesh,
    scratch_types=[
        pltpu.VMEM_SHARED((NPAD,), jnp.float32),
        pltpu.VMEM((CH,), jnp.int32),
        pltpu.VMEM((32,), jnp.int32),
        pltpu.VMEM((CH,), jnp.float32),
        pltpu.VMEM((32,), jnp.float32),
        pltpu.VMEM((640,), jnp.float32),
    ],
)
def _sc_degree(edges_hbm, out_hbm, acc, idx, idx_t, ones, ones_t, zb):
    c = lax.axis_index("c")
    s = lax.axis_index("s")
    _fill(ones, CH, 1.0)
    _fill(ones_t, 32, 1.0)
    _fill(zb, 640, 0.0)
    pltpu.sync_copy(zb, acc.at[pl.ds(pl.multiple_of(s * 640, 8), 640)])
    plsc.subcore_barrier()

    ept = (2 * E) // (NC * NS)
    base = c * E + s * ept
    nfull = ept // CH
    tail = ept - nfull * CH

    def chunk(j, _):
        cb = pl.multiple_of(base + j * CH, 8)
        pltpu.sync_copy(edges_hbm.at[pl.ds(cb, CH)], idx)
        pltpu.sync_copy(ones, acc.at[idx], add=True)
        return 0
    lax.fori_loop(0, nfull, chunk, 0)
    tb = pl.multiple_of(base + nfull * CH, 8)
    pltpu.sync_copy(edges_hbm.at[pl.ds(tb, tail)], idx_t)
    pltpu.sync_copy(ones_t, acc.at[idx_t], add=True)

    plsc.subcore_barrier()
    off = pl.multiple_of(c * NPAD + s * 640, 8)
    pltpu.sync_copy(acc.at[pl.ds(pl.multiple_of(s * 640, 8), 640)],
                    out_hbm.at[pl.ds(off, 640)])


@functools.partial(
    pl.kernel,
    out_type=jax.ShapeDtypeStruct((2 * NPAD, D), jnp.float32),
    mesh=_mesh,
    scratch_types=[
        pltpu.VMEM_SHARED((NPAD, D), jnp.float32),
        pltpu.VMEM((CH,), jnp.int32),
        pltpu.VMEM((CH,), jnp.int32),
        pltpu.VMEM((CH, D), jnp.float32),
        pltpu.VMEM((TAIL,), jnp.int32),
        pltpu.VMEM((TAIL,), jnp.int32),
        pltpu.VMEM((TAIL, D), jnp.float32),
        pltpu.VMEM((CH, D), jnp.float32),
        pltpu.SemaphoreType.DMA,
    ],
)
def _sc_agg(t_hbm, src_hbm, dst_hbm, out_hbm, acc, si, di, rows,
            si_t, di_t, rows_t, zrows, sem):
    c = lax.axis_index("c")
    s = lax.axis_index("s")

    def zfill(i, _):
        for j in range(D // 16):
            zrows[i, pl.ds(j * 16, 16)] = jnp.zeros((16,), jnp.float32)
        return 0
    lax.fori_loop(0, CH, zfill, 0)
    for k in range(640 // CH):
        pltpu.sync_copy(zrows, acc.at[pl.ds(pl.multiple_of(s * 640 + k * CH, 8), CH), :])
    plsc.subcore_barrier()

    base = c * EPC + s * EPT

    def chunk(j, _):
        cb = pl.multiple_of(base + j * CH, 8)
        pltpu.sync_copy(src_hbm.at[pl.ds(cb, CH)], si)
        pltpu.sync_copy(dst_hbm.at[pl.ds(cb, CH)], di)
        pltpu.async_copy(t_hbm.at[si], rows, sem).wait()
        pltpu.sync_copy(rows, acc.at[di], add=True)
        return 0
    lax.fori_loop(0, NFULL, chunk, 0)

    tb = pl.multiple_of(base + NFULL * CH, 8)
    pltpu.sync_copy(src_hbm.at[pl.ds(tb, TAIL)], si_t)
    pltpu.sync_copy(dst_hbm.at[pl.ds(tb, TAIL)], di_t)
    pltpu.async_copy(t_hbm.at[si_t], rows_t, sem).wait()
    pltpu.sync_copy(rows_t, acc.at[di_t], add=True)

    plsc.subcore_barrier()
    off = pl.multiple_of(c * NPAD + s * 640, 8)
    pltpu.sync_copy(acc.at[pl.ds(pl.multiple_of(s * 640, 8), 640), :],
                    out_hbm.at[pl.ds(off, 640), :])


@functools.partial(
    pl.kernel,
    out_type=(jax.ShapeDtypeStruct((E,), jnp.float32),
              jax.ShapeDtypeStruct((E,), jnp.float32)),
    mesh=_mesh,
    scratch_types=[
        pltpu.VMEM((CH,), jnp.int32),
        pltpu.VMEM((CH,), jnp.int32),
        pltpu.VMEM((CH,), jnp.float32),
        pltpu.VMEM((CH,), jnp.float32),
        pltpu.VMEM((CH,), jnp.float32),
        pltpu.VMEM((TAIL,), jnp.int32),
        pltpu.VMEM((TAIL,), jnp.int32),
        pltpu.VMEM((TAIL,), jnp.float32),
        pltpu.VMEM((TAIL,), jnp.float32),
        pltpu.VMEM((TAIL,), jnp.float32),
        pltpu.SemaphoreType.DMA,
        pltpu.SemaphoreType.DMA,
    ],
)
def _sc_scores(sa_hbm, sb_hbm, src_hbm, dst_hbm, nsrc_hbm, ndst_hbm,
               pos_hbm, neg_hbm, ia, ib, ga, gb, ob,
               ia_t, ib_t, ga_t, gb_t, ob_t, sem_a, sem_b):
    c = lax.axis_index("c")
    s = lax.axis_index("s")
    w = c * NS + s
    base = w * EPT

    def run(a_hbm, b_hbm, out_hbm):
        def chunk(j, _):
            cb = pl.multiple_of(base + j * CH, 8)
            pltpu.sync_copy(a_hbm.at[pl.ds(cb, CH)], ia)
            pltpu.sync_copy(b_hbm.at[pl.ds(cb, CH)], ib)
            pltpu.async_copy(sa_hbm.at[ia], ga, sem_a).wait()
            pltpu.async_copy(sb_hbm.at[ib], gb, sem_b).wait()
            for k in range(CH // 16):
                ob[pl.ds(k * 16, 16)] = ga[pl.ds(k * 16, 16)] + gb[pl.ds(k * 16, 16)]
            pltpu.sync_copy(ob, out_hbm.at[pl.ds(cb, CH)])
            return 0
        lax.fori_loop(0, NFULL, chunk, 0)
        tb = pl.multiple_of(base + NFULL * CH, 8)
        pltpu.sync_copy(a_hbm.at[pl.ds(tb, TAIL)], ia_t)
        pltpu.sync_copy(b_hbm.at[pl.ds(tb, TAIL)], ib_t)
        pltpu.async_copy(sa_hbm.at[ia_t], ga_t, sem_a).wait()
        pltpu.async_copy(sb_hbm.at[ib_t], gb_t, sem_b).wait()
        ob_t[pl.ds(0, 16)] = ga_t[pl.ds(0, 16)] + gb_t[pl.ds(0, 16)]
        pltpu.sync_copy(ob_t, out_hbm.at[pl.ds(tb, TAIL)])

    run(src_hbm, dst_hbm, pos_hbm)
    run(nsrc_hbm, ndst_hbm, neg_hbm)


def _norms(cnt_ref):
    cnt = cnt_ref[...]
    nrm = jnp.where(cnt > 0.0, lax.rsqrt(cnt), 0.0)
    return nrm[:N, 0:1], nrm[:N, 1:2]


def _tc0_body(cnt_ref, x_ref, t0_ref):
    ns, _ = _norms(cnt_ref)
    t0_ref[...] = x_ref[...] * ns


def _tc1_body(cnt_ref, agg_ref, w1_ref, b1_ref, t1_ref):
    ns, nd = _norms(cnt_ref)
    agg = (agg_ref[0, :N, :] + agg_ref[1, :N, :]) * nd
    h1 = jax.nn.relu(jnp.dot(agg, w1_ref[...],
                             preferred_element_type=jnp.float32) + b1_ref[...])
    t1_ref[...] = h1 * ns


def _tc2_body(cnt_ref, agg_ref, w2_ref, b2_ref, w3_ref, t2_ref):
    ns, nd = _norms(cnt_ref)
    agg = (agg_ref[0, :N, :] + agg_ref[1, :N, :]) * nd
    h2 = jax.nn.relu(jnp.dot(agg, w2_ref[...],
                             preferred_element_type=jnp.float32) + b2_ref[...])
    t2_ref[...] = jnp.dot(h2 * ns, w3_ref[...], preferred_element_type=jnp.float32)


def _tc3_body(cnt_ref, agg_ref, b3_ref, wa_ref, wb_ref, blp_ref,
              h3_ref, sa_ref, sb_ref):
    _, nd = _norms(cnt_ref)
    h3 = (agg_ref[0, :N, :] + agg_ref[1, :N, :]) * nd + b3_ref[...]
    h3_ref[...] = h3
    sa_ref[...] = jnp.dot(h3, wa_ref[...], preferred_element_type=jnp.float32) + blp_ref[...]
    sb_ref[...] = jnp.dot(h3, wb_ref[...], preferred_element_type=jnp.float32)


_f32 = jnp.float32

_tc0 = pl.pallas_call(_tc0_body, out_shape=jax.ShapeDtypeStruct((N, D), _f32))
_tc1 = pl.pallas_call(_tc1_body, out_shape=jax.ShapeDtypeStruct((N, D), _f32))
_tc2 = pl.pallas_call(_tc2_body, out_shape=jax.ShapeDtypeStruct((N, D), _f32))
_tc3 = pl.pallas_call(
    _tc3_body,
    out_shape=(jax.ShapeDtypeStruct((N, D), _f32),
               jax.ShapeDtypeStruct((N, 1), _f32),
               jax.ShapeDtypeStruct((N, 1), _f32)),
)


def kernel(x, edge_index, neg_src, neg_dst, W1, b1, W2, b2, W3, b3, Wlp, blp):
    src = edge_index[0]
    dst = edge_index[1]

    edges_all = jnp.concatenate([src, dst])
    counts = _sc_degree(edges_all)
    cnt = counts.reshape(2, NPAD).T

    t0 = _tc0(cnt, x)
    agg1 = _sc_agg(t0, src, dst).reshape(2, NPAD, D)
    t1 = _tc1(cnt, agg1, W1, b1.reshape(1, D))
    agg2 = _sc_agg(t1, src, dst).reshape(2, NPAD, D)
    t2 = _tc2(cnt, agg2, W2, b2.reshape(1, 2 * D), W3)
    agg3 = _sc_agg(t2, src, dst).reshape(2, NPAD, D)
    h3, sa, sb = _tc3(cnt, agg3, b3.reshape(1, D),
                      Wlp[:D], Wlp[D:], blp.reshape(1, 1))

    pos, neg = _sc_scores(sa.reshape(N), sb.reshape(N),
                          src, dst, neg_src, neg_dst)
    return (h3, pos, neg)
